# baseline (speedup 1.0000x reference)
import functools

import jax
import jax.numpy as jnp
from jax import lax
from jax.experimental import pallas as pl
from jax.experimental.pallas import tpu as pltpu
from jax.experimental.pallas import tpu_sc as plsc

N = 10000
E = 640000
IN_DIM = 3
HID = 128
HEADS = 4
OUT = HID // HEADS
L = 3
G = 16

NP = 10240
HH = HID // 2
CK = 64
CH = 626
EPT = CH * CK
EPAD = 16 * EPT
RPT = NP // 16


def _leaky(v):
    return jnp.where(v >= 0, v, v * 0.2)


def _mhat_row(alsd):
    mx = jnp.max(alsd, axis=0, keepdims=True)
    ms = mx[:, :HEADS] + mx[:, HEADS:]
    m = _leaky(ms)
    return jnp.concatenate([m, jnp.zeros((1, 12), jnp.float32)], axis=1)


def _tc_prep_body(x_ref, wemb_ref, bemb_ref, s8_ref, w_ref,
                  hpl_ref, hpr_ref, alsd_ref, mh_ref):
    h = jnp.dot(x_ref[...], wemb_ref[...],
                preferred_element_type=jnp.float32) + bemb_ref[...]
    hp = jnp.dot(h, w_ref[...], preferred_element_type=jnp.float32)
    alsd = jnp.dot(hp, s8_ref[...], preferred_element_type=jnp.float32)
    hpl_ref[...] = hp[:, :HH]
    hpr_ref[...] = hp[:, HH:]
    alsd_ref[...] = alsd
    mh_ref[...] = _mhat_row(alsd)


def _combine_ln(acc0, acc1, r, bias, gamma, beta):
    den = jnp.concatenate([acc0[:, HH:HH + 2], acc1[:, HH:HH + 2]], axis=1)
    den_exp = jnp.dot(den, r, preferred_element_type=jnp.float32)
    acc = jnp.concatenate([acc0[:, :HH], acc1[:, :HH]], axis=1)
    out = acc / (den_exp + 1e-16) + bias
    mu = jnp.mean(out, axis=1, keepdims=True)
    d = out - mu
    var = jnp.mean(d * d, axis=1, keepdims=True)
    return d * jax.lax.rsqrt(var + 1e-5) * gamma + beta


def _tc_mid_body(acc0_ref, acc1_ref, r_ref,
                 bias_ref, gamma_ref, beta_ref, s8_ref, w_ref,
                 hpl_ref, hpr_ref, alsd_ref, mh_ref):
    h = _combine_ln(acc0_ref[...], acc1_ref[...],
                    r_ref[...], bias_ref[...], gamma_ref[...], beta_ref[...])
    hp = jnp.dot(h, w_ref[...], preferred_element_type=jnp.float32)
    alsd = jnp.dot(hp, s8_ref[...], preferred_element_type=jnp.float32)
    hpl_ref[...] = hp[:, :HH]
    hpr_ref[...] = hp[:, HH:]
    alsd_ref[...] = alsd
    mh_ref[...] = _mhat_row(alsd)


def _tc_final_body(acc0_ref, acc1_ref, r_ref,
                   bias_ref, gamma_ref, beta_ref, batch_ref,
                   h_ref, gr_ref):
    h = _combine_ln(acc0_ref[...], acc1_ref[...],
                    r_ref[...], bias_ref[...], gamma_ref[...], beta_ref[...])
    h_ref[...] = h
    gids = jax.lax.broadcasted_iota(jnp.int32, (G, NP), 0)
    onehot = (gids == batch_ref[...]).astype(jnp.float32)
    sums = jnp.dot(onehot, h, preferred_element_type=jnp.float32)
    cnt = jnp.sum(onehot, axis=1, keepdims=True)
    gr_ref[...] = sums / jnp.maximum(cnt, 1.0)


_tc_prep = pl.pallas_call(
    _tc_prep_body,
    out_shape=[
        jax.ShapeDtypeStruct((NP, HH), jnp.float32),
        jax.ShapeDtypeStruct((NP, HH), jnp.float32),
        jax.ShapeDtypeStruct((NP, 8), jnp.float32),
        jax.ShapeDtypeStruct((1, 16), jnp.float32),
    ],
)

_tc_mid = pl.pallas_call(
    _tc_mid_body,
    out_shape=[
        jax.ShapeDtypeStruct((NP, HH), jnp.float32),
        jax.ShapeDtypeStruct((NP, HH), jnp.float32),
        jax.ShapeDtypeStruct((NP, 8), jnp.float32),
        jax.ShapeDtypeStruct((1, 16), jnp.float32),
    ],
)

_tc_final = pl.pallas_call(
    _tc_final_body,
    out_shape=[
        jax.ShapeDtypeStruct((NP, HID), jnp.float32),
        jax.ShapeDtypeStruct((G, HID), jnp.float32),
    ],
)


def _sc_edge_body(src_hbm, dst_hbm, al0_hbm, al1_hbm, mh_hbm,
                  hpl_hbm, hpr_hbm,
                  z80_hbm,
                  acc_out,
                  al_v, mh_v,
                  src_a, src_b, dst_a, dst_b, dsc_a, dsc_b,
                  rows_a, rows_b, half_a, half_b,
                  ss_a, ss_b, sg_a, sg_b, sw_a, sw_b,
                  acc_sh):
    cid = lax.axis_index("c")
    sid = lax.axis_index("s")

    @pl.when(cid == 0)
    def _():
        pltpu.sync_copy(al0_hbm, al_v)

    @pl.when(cid == 1)
    def _():
        pltpu.sync_copy(al1_hbm, al_v)

    pltpu.sync_copy(mh_hbm, mh_v)

    r0 = sid * RPT
    for j in range(RPT // 128):
        pltpu.sync_copy(z80_hbm, acc_sh.at[pl.ds(r0 + j * 128, 128)])
    pltpu.sync_copy(z80_hbm.at[pl.ds(0, CK)], half_a)
    pltpu.sync_copy(z80_hbm.at[pl.ds(0, CK)], half_b)
    plsc.subcore_barrier()

    iota = lax.iota(jnp.int32, 16)
    mh_all = [mh_v[pl.ds(h * 16, 16)] for h in range(4)]
    mh = [jnp.where(cid == 0, mh_all[hh], mh_all[2 + hh]) for hh in range(2)]
    ebase = sid * EPT
    slot = {0: (src_a, dst_a, dsc_a, rows_a, half_a, ss_a, sg_a, sw_a),
            1: (src_b, dst_b, dsc_b, rows_b, half_b, ss_b, sg_b, sw_b)}

    def issue_srcdst(c, s):
        src_v, dst_v = slot[s][0], slot[s][1]
        off = ebase + c * CK
        pltpu.async_copy(src_hbm.at[pl.ds(off, CK)], src_v, slot[s][5])
        pltpu.async_copy(dst_hbm.at[pl.ds(off, CK)], dst_v, slot[s][5])

    def wait_srcdst(s):
        src_v, dst_v = slot[s][0], slot[s][1]
        pltpu.make_async_copy(src_hbm.at[pl.ds(0, CK)], src_v, slot[s][5]).wait()
        pltpu.make_async_copy(dst_hbm.at[pl.ds(0, CK)], dst_v, slot[s][5]).wait()

    def compute(c, s):
        src_v, dst_v, dsc_v, rows_v, half_v = slot[s][:5]
        off = ebase + c * CK

        def group(g, carry2):
            k0 = g * 16
            s16 = src_v[pl.ds(k0, 16)]
            d16 = dst_v[pl.ds(k0, 16)]
            dsc_v[pl.ds(k0, 16)] = d16
            r16 = k0 + iota
            valid = (off + r16) < E
            for hh in range(2):
                a_s = plsc.load_gather(al_v, [s16 * 4 + hh])
                a_d = plsc.load_gather(al_v, [d16 * 4 + (2 + hh)])
                sv = a_s + a_d
                ev = jnp.where(sv >= 0, sv, sv * 0.2)
                ex = jnp.exp(ev - mh[hh])
                ex = jnp.where(valid, ex, 0.0)
                plsc.store_scatter(half_v,
                                   [r16, jnp.full((16,), HH + hh, jnp.int32)],
                                   ex)
                for cc in range(hh * OUT, (hh + 1) * OUT):
                    fc = jnp.full((16,), cc, jnp.int32)
                    col = plsc.load_gather(rows_v, [r16, fc])
                    plsc.store_scatter(half_v, [r16, fc], col * ex)
            return carry2

        lax.fori_loop(0, CK // 16, group, 0)

    def wait_scatter(s):
        dsc_v, _, half_v = slot[s][2:5]
        pltpu.make_async_copy(half_v, acc_sh.at[dsc_v], slot[s][7]).wait()

    def step(c, cur, nxt):
        @pl.when(c + 1 < CH)
        def _():
            wait_srcdst(nxt)

            @pl.when(cid == 0)
            def _():
                pltpu.async_copy(hpl_hbm.at[slot[nxt][0]], slot[nxt][3],
                                 slot[nxt][6])

            @pl.when(cid == 1)
            def _():
                pltpu.async_copy(hpr_hbm.at[slot[nxt][0]], slot[nxt][3],
                                 slot[nxt][6])
        pltpu.make_async_copy(hpl_hbm.at[slot[cur][0]], slot[cur][3],
                              slot[cur][6]).wait()

        @pl.when(c >= 2)
        def _():
            wait_scatter(cur)
        compute(c, cur)
        pltpu.async_copy(slot[cur][4], acc_sh.at[slot[cur][2]], slot[cur][7],
                         add=True)

        @pl.when(c + 2 < CH)
        def _():
            issue_srcdst(c + 2, cur)

    issue_srcdst(0, 0)
    wait_srcdst(0)

    @pl.when(cid == 0)
    def _():
        pltpu.async_copy(hpl_hbm.at[src_a], rows_a, sg_a)

    @pl.when(cid == 1)
    def _():
        pltpu.async_copy(hpr_hbm.at[src_a], rows_a, sg_a)

    issue_srcdst(1, 1)

    def pair(i, carry):
        step(2 * i, 0, 1)
        step(2 * i + 1, 1, 0)
        return carry

    lax.fori_loop(0, CH // 2, pair, 0)
    wait_scatter(0)
    wait_scatter(1)
    plsc.subcore_barrier()

    out_r0 = cid * NP + r0
    pltpu.sync_copy(acc_sh.at[pl.ds(r0, RPT)], acc_out.at[pl.ds(out_r0, RPT)])


_SC_PARAMS = pltpu.CompilerParams(needs_layout_passes=False,
                                  use_tc_tiling_on_sc=False)


@functools.cache
def _build_sc_edge():
  return pl.kernel(
    _sc_edge_body,
    compiler_params=_SC_PARAMS,
    out_type=jax.ShapeDtypeStruct((2 * NP, HH + 16), jnp.float32),
    mesh=plsc.VectorSubcoreMesh(core_axis_name="c", subcore_axis_name="s",
                                num_cores=2, num_subcores=16),
    scratch_types=(
        [pltpu.VMEM((NP * 4,), jnp.float32),
         pltpu.VMEM((64,), jnp.float32)]
        + [pltpu.VMEM((CK,), jnp.int32)] * 6
        + [pltpu.VMEM((CK, HH), jnp.float32)] * 2
        + [pltpu.VMEM((CK, HH + 16), jnp.float32)] * 2
        + [pltpu.SemaphoreType.DMA] * 6
        + [pltpu.VMEM_SHARED((NP, HH + 16), jnp.float32)]
    ),
  )


def kernel(x, edge_index, batch, W_emb, b_emb, Ws, a_srcs, a_dsts, biases,
           gammas, betas):
    src = jnp.pad(edge_index[0], (0, EPAD - E))
    dst = jnp.pad(edge_index[1], (0, EPAD - E))
    x_pad = jnp.pad(x, ((0, NP - N), (0, 0)))
    batch_pad = jnp.pad(batch, (0, NP - N), constant_values=G)[None, :]

    eye = jnp.eye(HEADS, dtype=jnp.float32)
    blk = jnp.repeat(eye, OUT, axis=0)
    s8 = jnp.concatenate(
        [blk[None] * a_srcs.reshape(L, HID)[:, :, None],
         blk[None] * a_dsts.reshape(L, HID)[:, :, None]],
        axis=2)
    r_exp = jnp.repeat(eye, OUT, axis=1)

    z80 = jnp.zeros((128, HH + 16), jnp.float32)

    sc_edge = _build_sc_edge()
    hpl, hpr, alsd, mh = _tc_prep(x_pad, W_emb, b_emb[None, :], s8[0], Ws[0])
    for l in range(L):
        mh64 = jnp.repeat(mh.reshape(-1)[:HEADS], 16)
        al0 = alsd[:, [0, 1, 4, 5]].reshape(-1)
        al1 = alsd[:, [2, 3, 6, 7]].reshape(-1)
        acc = sc_edge(src, dst, al0, al1, mh64, hpl, hpr, z80)
        acc0, acc1 = acc[:NP], acc[NP:]
        if l < L - 1:
            hpl, hpr, alsd, mh = _tc_mid(acc0, acc1, r_exp,
                                   biases[l][None, :],
                                   gammas[l][None, :],
                                   betas[l][None, :], s8[l + 1],
                                   Ws[l + 1])
        else:
            h_full, graph_repr = _tc_final(acc0, acc1, r_exp,
                                           biases[l][None, :],
                                           gammas[l][None, :],
                                           betas[l][None, :], batch_pad)
    return h_full[:N], graph_repr


# speedup vs baseline: 1.0018x; 1.0018x over previous
import functools

import jax
import jax.numpy as jnp
from jax import lax
from jax.experimental import pallas as pl
from jax.experimental.pallas import tpu as pltpu
from jax.experimental.pallas import tpu_sc as plsc

N = 10000
E = 640000
IN_DIM = 3
HID = 128
HEADS = 4
OUT = HID // HEADS
L = 3
G = 16

NP = 10240
HH = HID // 2
CK = 64
CH = 626
EPT = CH * CK
EPAD = 16 * EPT
RPT = NP // 16


def _leaky(v):
    return jnp.where(v >= 0, v, v * 0.2)


def _mhat_row(alsd):
    mx = jnp.max(alsd, axis=0, keepdims=True)
    ms = mx[:, :HEADS] + mx[:, HEADS:]
    m = _leaky(ms)
    return jnp.concatenate([m, jnp.zeros((1, 12), jnp.float32)], axis=1)


def _tc_prep_body(x_ref, wemb_ref, bemb_ref, s8_ref, w_ref,
                  hp_ref, alsd_ref, mh_ref):
    h = jnp.dot(x_ref[...], wemb_ref[...],
                preferred_element_type=jnp.float32) + bemb_ref[...]
    hp = jnp.dot(h, w_ref[...], preferred_element_type=jnp.float32)
    alsd = jnp.dot(hp, s8_ref[...], preferred_element_type=jnp.float32)
    hp_ref[...] = hp
    alsd_ref[...] = alsd
    mh_ref[...] = _mhat_row(alsd)


def _combine_ln(acc0, acc1, r, bias, gamma, beta):
    den = jnp.concatenate([acc0[:, HH:HH + 2], acc1[:, HH:HH + 2]], axis=1)
    den_exp = jnp.dot(den, r, preferred_element_type=jnp.float32)
    acc = jnp.concatenate([acc0[:, :HH], acc1[:, :HH]], axis=1)
    out = acc / (den_exp + 1e-16) + bias
    mu = jnp.mean(out, axis=1, keepdims=True)
    d = out - mu
    var = jnp.mean(d * d, axis=1, keepdims=True)
    return d * jax.lax.rsqrt(var + 1e-5) * gamma + beta


def _tc_mid_body(acc0_ref, acc1_ref, r_ref,
                 bias_ref, gamma_ref, beta_ref, s8_ref, w_ref,
                 hp_ref, alsd_ref, mh_ref):
    h = _combine_ln(acc0_ref[...], acc1_ref[...],
                    r_ref[...], bias_ref[...], gamma_ref[...], beta_ref[...])
    hp = jnp.dot(h, w_ref[...], preferred_element_type=jnp.float32)
    alsd = jnp.dot(hp, s8_ref[...], preferred_element_type=jnp.float32)
    hp_ref[...] = hp
    alsd_ref[...] = alsd
    mh_ref[...] = _mhat_row(alsd)


def _tc_final_body(acc0_ref, acc1_ref, r_ref,
                   bias_ref, gamma_ref, beta_ref, batch_ref,
                   h_ref, gr_ref):
    h = _combine_ln(acc0_ref[...], acc1_ref[...],
                    r_ref[...], bias_ref[...], gamma_ref[...], beta_ref[...])
    h_ref[...] = h
    gids = jax.lax.broadcasted_iota(jnp.int32, (G, NP), 0)
    onehot = (gids == batch_ref[...]).astype(jnp.float32)
    sums = jnp.dot(onehot, h, preferred_element_type=jnp.float32)
    cnt = jnp.sum(onehot, axis=1, keepdims=True)
    gr_ref[...] = sums / jnp.maximum(cnt, 1.0)


_tc_prep = pl.pallas_call(
    _tc_prep_body,
    out_shape=[
        jax.ShapeDtypeStruct((NP, HID), jnp.float32),
        jax.ShapeDtypeStruct((NP, 8), jnp.float32),
        jax.ShapeDtypeStruct((1, 16), jnp.float32),
    ],
)

_tc_mid = pl.pallas_call(
    _tc_mid_body,
    out_shape=[
        jax.ShapeDtypeStruct((NP, HID), jnp.float32),
        jax.ShapeDtypeStruct((NP, 8), jnp.float32),
        jax.ShapeDtypeStruct((1, 16), jnp.float32),
    ],
)

_tc_final = pl.pallas_call(
    _tc_final_body,
    out_shape=[
        jax.ShapeDtypeStruct((NP, HID), jnp.float32),
        jax.ShapeDtypeStruct((G, HID), jnp.float32),
    ],
)


def _sc_edge_body(src_hbm, dst_hbm, al0_hbm, al1_hbm, mh_hbm, hp_hbm,
                  z80_hbm,
                  acc_out,
                  al_v, mh_v,
                  src_a, src_b, dst_a, dst_b, dsc_a, dsc_b,
                  rows_a, rows_b, half_a, half_b,
                  ss_a, ss_b, sg_a, sg_b, sw_a, sw_b,
                  acc_sh):
    cid = lax.axis_index("c")
    sid = lax.axis_index("s")

    @pl.when(cid == 0)
    def _():
        pltpu.sync_copy(al0_hbm, al_v)

    @pl.when(cid == 1)
    def _():
        pltpu.sync_copy(al1_hbm, al_v)

    pltpu.sync_copy(mh_hbm, mh_v)

    r0 = sid * RPT
    for j in range(RPT // 128):
        pltpu.sync_copy(z80_hbm, acc_sh.at[pl.ds(r0 + j * 128, 128)])
    pltpu.sync_copy(z80_hbm.at[pl.ds(0, CK)], half_a)
    pltpu.sync_copy(z80_hbm.at[pl.ds(0, CK)], half_b)
    plsc.subcore_barrier()

    iota = lax.iota(jnp.int32, 16)
    mh_all = [mh_v[pl.ds(h * 16, 16)] for h in range(4)]
    mh = [jnp.where(cid == 0, mh_all[hh], mh_all[2 + hh]) for hh in range(2)]
    ebase = sid * EPT
    slot = {0: (src_a, dst_a, dsc_a, rows_a, half_a, ss_a, sg_a, sw_a),
            1: (src_b, dst_b, dsc_b, rows_b, half_b, ss_b, sg_b, sw_b)}

    def issue_srcdst(c, s):
        src_v, dst_v = slot[s][0], slot[s][1]
        off = ebase + c * CK
        pltpu.async_copy(src_hbm.at[pl.ds(off, CK)], src_v, slot[s][5])
        pltpu.async_copy(dst_hbm.at[pl.ds(off, CK)], dst_v, slot[s][5])

    def wait_srcdst(s):
        src_v, dst_v = slot[s][0], slot[s][1]
        pltpu.make_async_copy(src_hbm.at[pl.ds(0, CK)], src_v, slot[s][5]).wait()
        pltpu.make_async_copy(dst_hbm.at[pl.ds(0, CK)], dst_v, slot[s][5]).wait()

    def compute(c, s):
        src_v, dst_v, dsc_v, rows_v, half_v = slot[s][:5]
        off = ebase + c * CK

        def group(g, carry2):
            k0 = g * 16
            s16 = src_v[pl.ds(k0, 16)]
            d16 = dst_v[pl.ds(k0, 16)]
            dsc_v[pl.ds(k0, 16)] = d16
            r16 = k0 + iota
            valid = (off + r16) < E
            cbase = cid * HH
            for hh in range(2):
                a_s = plsc.load_gather(al_v, [s16 * 4 + hh])
                a_d = plsc.load_gather(al_v, [d16 * 4 + (2 + hh)])
                sv = a_s + a_d
                ev = jnp.where(sv >= 0, sv, sv * 0.2)
                ex = jnp.exp(ev - mh[hh])
                ex = jnp.where(valid, ex, 0.0)
                plsc.store_scatter(half_v,
                                   [r16, jnp.full((16,), HH + hh, jnp.int32)],
                                   ex)
                for cc in range(hh * OUT, (hh + 1) * OUT):
                    fcg = jnp.full((16,), cc, jnp.int32) + cbase
                    fcl = jnp.full((16,), cc, jnp.int32)
                    col = plsc.load_gather(rows_v, [r16, fcg])
                    plsc.store_scatter(half_v, [r16, fcl], col * ex)
            return carry2

        lax.fori_loop(0, CK // 16, group, 0)

    def wait_scatter(s):
        dsc_v, _, half_v = slot[s][2:5]
        pltpu.make_async_copy(half_v, acc_sh.at[dsc_v], slot[s][7]).wait()

    def step(c, cur, nxt):
        @pl.when(c + 1 < CH)
        def _():
            wait_srcdst(nxt)
            pltpu.async_copy(hp_hbm.at[slot[nxt][0]], slot[nxt][3],
                             slot[nxt][6])
        pltpu.make_async_copy(hp_hbm.at[slot[cur][0]], slot[cur][3],
                              slot[cur][6]).wait()

        @pl.when(c >= 2)
        def _():
            wait_scatter(cur)
        compute(c, cur)
        pltpu.async_copy(slot[cur][4], acc_sh.at[slot[cur][2]], slot[cur][7],
                         add=True)

        @pl.when(c + 2 < CH)
        def _():
            issue_srcdst(c + 2, cur)

    issue_srcdst(0, 0)
    wait_srcdst(0)
    pltpu.async_copy(hp_hbm.at[src_a], rows_a, sg_a)
    issue_srcdst(1, 1)

    def pair(i, carry):
        step(2 * i, 0, 1)
        step(2 * i + 1, 1, 0)
        return carry

    lax.fori_loop(0, CH // 2, pair, 0)
    wait_scatter(0)
    wait_scatter(1)
    plsc.subcore_barrier()

    out_r0 = cid * NP + r0
    pltpu.sync_copy(acc_sh.at[pl.ds(r0, RPT)], acc_out.at[pl.ds(out_r0, RPT)])


_SC_PARAMS = pltpu.CompilerParams(needs_layout_passes=False,
                                  use_tc_tiling_on_sc=False)


@functools.cache
def _build_sc_edge():
  return pl.kernel(
    _sc_edge_body,
    compiler_params=_SC_PARAMS,
    out_type=jax.ShapeDtypeStruct((2 * NP, HH + 16), jnp.float32),
    mesh=plsc.VectorSubcoreMesh(core_axis_name="c", subcore_axis_name="s",
                                num_cores=2, num_subcores=16),
    scratch_types=(
        [pltpu.VMEM((NP * 4,), jnp.float32),
         pltpu.VMEM((64,), jnp.float32)]
        + [pltpu.VMEM((CK,), jnp.int32)] * 6
        + [pltpu.VMEM((CK, HID), jnp.float32)] * 2
        + [pltpu.VMEM((CK, HH + 16), jnp.float32)] * 2
        + [pltpu.SemaphoreType.DMA] * 6
        + [pltpu.VMEM_SHARED((NP, HH + 16), jnp.float32)]
    ),
  )


def kernel(x, edge_index, batch, W_emb, b_emb, Ws, a_srcs, a_dsts, biases,
           gammas, betas):
    src = jnp.pad(edge_index[0], (0, EPAD - E))
    dst = jnp.pad(edge_index[1], (0, EPAD - E))
    x_pad = jnp.pad(x, ((0, NP - N), (0, 0)))
    batch_pad = jnp.pad(batch, (0, NP - N), constant_values=G)[None, :]

    eye = jnp.eye(HEADS, dtype=jnp.float32)
    blk = jnp.repeat(eye, OUT, axis=0)
    s8 = jnp.concatenate(
        [blk[None] * a_srcs.reshape(L, HID)[:, :, None],
         blk[None] * a_dsts.reshape(L, HID)[:, :, None]],
        axis=2)
    r_exp = jnp.repeat(eye, OUT, axis=1)

    z80 = jnp.zeros((128, HH + 16), jnp.float32)

    sc_edge = _build_sc_edge()
    hp, alsd, mh = _tc_prep(x_pad, W_emb, b_emb[None, :], s8[0], Ws[0])
    for l in range(L):
        mh64 = jnp.repeat(mh.reshape(-1)[:HEADS], 16)
        al0 = alsd[:, [0, 1, 4, 5]].reshape(-1)
        al1 = alsd[:, [2, 3, 6, 7]].reshape(-1)
        acc = sc_edge(src, dst, al0, al1, mh64, hp, z80)
        acc0, acc1 = acc[:NP], acc[NP:]
        if l < L - 1:
            hp, alsd, mh = _tc_mid(acc0, acc1, r_exp,
                                   biases[l][None, :],
                                   gammas[l][None, :],
                                   betas[l][None, :], s8[l + 1],
                                   Ws[l + 1])
        else:
            h_full, graph_repr = _tc_final(acc0, acc1, r_exp,
                                           biases[l][None, :],
                                           gammas[l][None, :],
                                           betas[l][None, :], batch_pad)
    return h_full[:N], graph_repr
